# baseline (speedup 1.0000x reference)
import jax
import jax.numpy as jnp
import numpy as np
from jax import lax
from jax.experimental import pallas as pl
from jax.experimental.pallas import tpu as pltpu
from jax.experimental.pallas import tpu_sc as plsc

B = 4096
L = 20
D = 32
H1 = 128
H2 = 64

NC = 2
NS = 16
NW = NC * NS
SPW = B // NW
RPW = SPW * L
CH = 128
NCHUNK = RPW // CH
NB = 4

_SIDX = (np.arange(NS)[:, None] * SPW
         + (np.arange(RPW) // L)[None, :]).astype(np.int32).reshape(
             NS, NCHUNK, CH)


def _pool_body(tq_hbm, tc_hbm, qidx_hbm, cidx_hbm, sidx_hbm, zeros_hbm,
               outq_hbm, outc_hbm,
               idx_v, sidx_v, rows_v, sharedq, sharedc, gsem, ssem):
    cid = lax.axis_index("c")
    sid = lax.axis_index("s")
    wid = sid * NC + cid
    base_s = wid * SPW
    base_i = wid * RPW

    pltpu.sync_copy(qidx_hbm.at[pl.ds(base_i, RPW)], idx_v.at[0])
    pltpu.sync_copy(cidx_hbm.at[pl.ds(base_i, RPW)], idx_v.at[1])
    pltpu.sync_copy(sidx_hbm.at[sid], sidx_v)

    pltpu.sync_copy(zeros_hbm, sharedq.at[pl.ds(sid * SPW, SPW)])
    pltpu.sync_copy(zeros_hbm, sharedc.at[pl.ds(sid * SPW, SPW)])

    work = [(0, j) for j in range(NCHUNK)] + [(1, j) for j in range(NCHUNK)]
    tbls = (tq_hbm, tc_hbm)
    accs = (sharedq, sharedc)

    def gather(k):
        tower, j = work[k]
        return pltpu.async_copy(
            tbls[tower].at[idx_v.at[tower, pl.ds(j * CH, CH)]],
            rows_v.at[k % NB], gsem)

    gd = [None] * len(work)
    sd = [None] * len(work)
    for k in range(NB):
        gd[k] = gather(k)
    for k in range(len(work)):
        tower, j = work[k]
        gd[k].wait()
        sd[k] = pltpu.async_copy(
            rows_v.at[k % NB], accs[tower].at[sidx_v.at[j]], ssem, add=True)
        nxt = k + NB
        if nxt < len(work):
            sd[k].wait()
            gd[nxt] = gather(nxt)
    for k in range(len(work) - NB, len(work)):
        sd[k].wait()

    pltpu.sync_copy(sharedq.at[pl.ds(sid * SPW, SPW)],
                    outq_hbm.at[pl.ds(base_s, SPW)])
    pltpu.sync_copy(sharedc.at[pl.ds(sid * SPW, SPW)],
                    outc_hbm.at[pl.ds(base_s, SPW)])


def _pooled_sc(table_q, table_c, qidx_flat, cidx_flat, sidx, zeros):
    mesh = plsc.VectorSubcoreMesh(core_axis_name="c", subcore_axis_name="s")
    return pl.kernel(
        _pool_body,
        out_type=(
            jax.ShapeDtypeStruct((B, D), jnp.float32),
            jax.ShapeDtypeStruct((B, D), jnp.float32),
        ),
        mesh=mesh,
        scratch_types=[
            pltpu.VMEM((2, RPW), jnp.int32),
            pltpu.VMEM((NCHUNK, CH), jnp.int32),
            pltpu.VMEM((NB, CH, D), jnp.float32),
            pltpu.VMEM_SHARED((NS * SPW, D), jnp.float32),
            pltpu.VMEM_SHARED((NS * SPW, D), jnp.float32),
            pltpu.SemaphoreType.DMA,
            pltpu.SemaphoreType.DMA,
        ],
        compiler_params=pltpu.CompilerParams(use_tc_tiling_on_sc=False),
    )(table_q, table_c, qidx_flat, cidx_flat, sidx, zeros)


def _mlp_body(xq_ref, xc_ref, wq1_ref, bq1_ref, wq2_ref, bq2_ref,
              wc1_ref, bc1_ref, wc2_ref, bc2_ref, oq_ref, oc_ref):
    hq = jnp.dot(xq_ref[...], wq1_ref[...], preferred_element_type=jnp.float32)
    hq = jnp.maximum(hq + bq1_ref[...], 0.0)
    oq = jnp.dot(hq, wq2_ref[...], preferred_element_type=jnp.float32)
    oq_ref[...] = jnp.maximum(oq + bq2_ref[...], 0.0)

    hc = jnp.dot(xc_ref[...], wc1_ref[...], preferred_element_type=jnp.float32)
    hc = jnp.maximum(hc + bc1_ref[...], 0.0)
    oc = jnp.dot(hc, wc2_ref[...], preferred_element_type=jnp.float32)
    oc_ref[...] = jnp.maximum(oc + bc2_ref[...], 0.0)


def _mlp_tc(pooled_q, pooled_c, Wq1, bq1, Wq2, bq2, Wc1, bc1, Wc2, bc2):
    BLK = 512
    grid = (B // BLK,)
    full = lambda r, c: pl.BlockSpec((r, c), lambda i: (0, 0))
    return pl.pallas_call(
        _mlp_body,
        grid=grid,
        in_specs=[
            pl.BlockSpec((BLK, D), lambda i: (i, 0)),
            pl.BlockSpec((BLK, D), lambda i: (i, 0)),
            full(D, H1), full(1, H1), full(H1, H2), full(1, H2),
            full(D, H1), full(1, H1), full(H1, H2), full(1, H2),
        ],
        out_specs=[
            pl.BlockSpec((BLK, H2), lambda i: (i, 0)),
            pl.BlockSpec((BLK, H2), lambda i: (i, 0)),
        ],
        out_shape=[
            jax.ShapeDtypeStruct((B, H2), jnp.float32),
            jax.ShapeDtypeStruct((B, H2), jnp.float32),
        ],
    )(pooled_q, pooled_c, Wq1, bq1, Wq2, bq2, Wc1, bc1, Wc2, bc2)


def kernel(query_indices, candidate_indices, table_q, table_c,
           Wq1, bq1, Wq2, bq2, Wc1, bc1, Wc2, bc2):
    qidx_flat = query_indices.astype(jnp.int32).reshape(B * L)
    cidx_flat = candidate_indices.astype(jnp.int32).reshape(B * L)
    sidx = jnp.asarray(_SIDX)
    zeros = jnp.zeros((SPW, D), jnp.float32)

    pooled_q, pooled_c = _pooled_sc(table_q, table_c, qidx_flat, cidx_flat,
                                    sidx, zeros)

    q, c = _mlp_tc(pooled_q, pooled_c,
                   Wq1, bq1[None, :], Wq2, bq2[None, :],
                   Wc1, bc1[None, :], Wc2, bc2[None, :])
    return q, c
